# Initial kernel scaffold; baseline (speedup 1.0000x reference)
import jax
import jax.numpy as jnp
from jax.experimental import pallas as pl


def kernel(node_emb, batch, edge_index, W1, b1, W2, b2):
    raise NotImplementedError("write your pallas kernel here")


# speedup vs baseline: 2.7801x; 2.7801x over previous
import functools

import jax
import jax.numpy as jnp
from jax.experimental import pallas as pl
from jax.experimental.pallas import tpu as pltpu

N = 10000
D = 128
G = 64
H = 256
NB = 10
BR = N // NB


def _pool_mlp_body(x_ref, ids_ref, w1_ref, b1_ref, w2_ref, b2_ref,
                   out_ref, pooled_scr):
    i = pl.program_id(0)

    @pl.when(i == 0)
    def _init():
        pooled_scr[...] = jnp.full((G, D), -jnp.inf, jnp.float32)

    x = x_ref[...]
    ids = ids_ref[0]
    gmin = jnp.min(ids)
    gmax = jnp.max(ids)

    def seg_body(g, carry):
        mask = ids == g
        seg = jnp.max(jnp.where(mask, x, -jnp.inf), axis=0)
        cur = pooled_scr[pl.ds(g, 1), :]
        pooled_scr[pl.ds(g, 1), :] = jnp.maximum(cur, seg[None, :])
        return carry

    jax.lax.fori_loop(gmin, gmax + 1, seg_body, 0)

    @pl.when(i == NB - 1)
    def _mlp():
        pooled = pooled_scr[...]
        h = jax.lax.dot_general(pooled, w1_ref[...],
                                (((1,), (1,)), ((), ())),
                                preferred_element_type=jnp.float32)
        h = jnp.maximum(h + b1_ref[...], 0.0)
        y = jax.lax.dot_general(h, w2_ref[...],
                                (((1,), (1,)), ((), ())),
                                preferred_element_type=jnp.float32)
        out_ref[...] = y + b2_ref[...]


@functools.partial(jax.jit, static_argnums=())
def _run(node_emb, batch3, W1, b1r, W2p, b2p):
    return pl.pallas_call(
        _pool_mlp_body,
        grid=(NB,),
        in_specs=[
            pl.BlockSpec((BR, D), lambda i: (i, 0)),
            pl.BlockSpec((1, BR, 1), lambda i: (i, 0, 0)),
            pl.BlockSpec((H, D), lambda i: (0, 0)),
            pl.BlockSpec((1, H), lambda i: (0, 0)),
            pl.BlockSpec((16, H), lambda i: (0, 0)),
            pl.BlockSpec((1, 16), lambda i: (0, 0)),
        ],
        out_specs=pl.BlockSpec((G, 16), lambda i: (0, 0)),
        out_shape=jax.ShapeDtypeStruct((G, 16), jnp.float32),
        scratch_shapes=[pltpu.VMEM((G, D), jnp.float32)],
    )(node_emb, batch3, W1, b1r, W2p, b2p)


def kernel(node_emb, batch, edge_index, W1, b1, W2, b2):
    T = W2.shape[0]
    batch3 = batch.reshape(NB, BR, 1)
    W2p = jnp.zeros((16, H), W2.dtype).at[:T].set(W2)
    b2p = jnp.zeros((1, 16), b2.dtype).at[0, :T].set(b2)
    b1r = b1.reshape(1, H)
    out = _run(node_emb, batch3, W1, b1r, W2p, b2p)
    return (out[:, :T], node_emb, edge_index)
